# baseline (speedup 1.0000x reference)
import functools

import jax
import jax.numpy as jnp
from jax import lax
from jax.experimental import pallas as pl
from jax.experimental.pallas import tpu as pltpu
from jax.experimental.pallas import tpu_sc as plsc

N = 10000
E = 320000
D = 128
R = 50

NP = 10240
EP = 327680
RP = 64
NC = 2
NS = 16
NT = NC * NS
ET = EP // NT
CH = 128
NCH = ET // CH
RPT = NP // NS
BLK = 512
NBLK = NP // BLK


def _proj_scores(i, h, wT_ref, b_ref, ai_ref, aj_ref, relp_ref, wrT_ref,
                 br_ref, ar_ref, hx_ref, si_ref, sj_ref, sr_ref, bm_ref):
    hx = jnp.dot(h, wT_ref[...], preferred_element_type=jnp.float32) + b_ref[...]
    hx_ref[...] = hx
    si = jnp.sum(hx * ai_ref[...], axis=1, keepdims=True)
    sj = jnp.sum(hx * aj_ref[...], axis=1, keepdims=True)
    si_ref[...] = si
    sj_ref[...] = sj
    bm_ref[...] = jnp.concatenate(
        [jnp.broadcast_to(jnp.max(si), (4, D)),
         jnp.broadcast_to(jnp.max(sj), (4, D))], axis=0)

    @pl.when(i == 0)
    def _():
        relp = jnp.dot(relp_ref[...], wrT_ref[...],
                       preferred_element_type=jnp.float32) + br_ref[...]
        sr_ref[...] = jnp.sum(relp * ar_ref[...], axis=1, keepdims=True)


def _tc_pre_body(h_ref, wT_ref, b_ref, ai_ref, aj_ref, relp_ref, wrT_ref,
                 br_ref, ar_ref, hx_ref, si_ref, sj_ref, sr_ref, bm_ref):
    i = pl.program_id(0)
    _proj_scores(i, h_ref[...], wT_ref, b_ref, ai_ref, aj_ref, relp_ref,
                 wrT_ref, br_ref, ar_ref, hx_ref, si_ref, sj_ref, sr_ref,
                 bm_ref)


def _agg_norm(h_ref, acc_ref, den_ref, g_ref, be_ref):
    acc = acc_ref[0] + acc_ref[1]
    den = jnp.sum(den_ref[...], axis=0)
    out = acc / (den[:, None] + 1e-16)
    hsum = h_ref[...] + out
    mean = jnp.mean(hsum, axis=1, keepdims=True)
    var = jnp.mean((hsum - mean) ** 2, axis=1, keepdims=True)
    hn = (hsum - mean) * lax.rsqrt(var + 1e-5) * g_ref[...] + be_ref[...]
    return jnp.maximum(hn, 0.0)


def _tc_mid_body(h_ref, acc_ref, den_ref, g_ref, be_ref, wT_ref, b_ref,
                 ai_ref, aj_ref, relp_ref, wrT_ref, br_ref, ar_ref,
                 h2_ref, hx_ref, si_ref, sj_ref, sr_ref, bm_ref):
    i = pl.program_id(0)
    h2 = _agg_norm(h_ref, acc_ref, den_ref, g_ref, be_ref)
    h2_ref[...] = h2
    _proj_scores(i, h2, wT_ref, b_ref, ai_ref, aj_ref, relp_ref, wrT_ref,
                 br_ref, ar_ref, hx_ref, si_ref, sj_ref, sr_ref, bm_ref)


def _tc_post_body(h_ref, acc_ref, den_ref, g_ref, be_ref, out_ref):
    out_ref[...] = _agg_norm(h_ref, acc_ref, den_ref, g_ref, be_ref)


def _row_spec():
    return pl.BlockSpec((BLK, D), lambda i: (i, 0))


def _full(shape):
    nd = len(shape)
    return pl.BlockSpec(shape, lambda i: (0,) * nd)


_SCORE_OUT_SHAPES = [
    jax.ShapeDtypeStruct((NP, D), jnp.float32),
    jax.ShapeDtypeStruct((NP, 1), jnp.float32),
    jax.ShapeDtypeStruct((NP, 1), jnp.float32),
    jax.ShapeDtypeStruct((RP, 1), jnp.float32),
    jax.ShapeDtypeStruct((NBLK * 8, D), jnp.float32),
]
_SCORE_OUT_SPECS = [
    _row_spec(),
    pl.BlockSpec((BLK, 1), lambda i: (i, 0)),
    pl.BlockSpec((BLK, 1), lambda i: (i, 0)),
    _full((RP, 1)),
    pl.BlockSpec((8, D), lambda i: (i, 0)),
]
_WEIGHT_SPECS = [
    _full((D, D)),
    _full((1, D)),
    _full((1, D)),
    _full((1, D)),
    _full((RP, D)),
    _full((D, D)),
    _full((1, D)),
    _full((1, D)),
]


def _tc_pre(h, wT, b, ai, aj, relp, wrT, br, ar):
    return pl.pallas_call(
        _tc_pre_body,
        grid=(NBLK,),
        in_specs=[_row_spec()] + _WEIGHT_SPECS,
        out_specs=_SCORE_OUT_SPECS,
        out_shape=_SCORE_OUT_SHAPES,
    )(h, wT, b, ai, aj, relp, wrT, br, ar)


_AGG_SPECS = [
    _row_spec(),
    pl.BlockSpec((NC, BLK, D), lambda i: (0, i, 0)),
    pl.BlockSpec((NT, BLK), lambda i: (0, i)),
    _full((1, D)),
    _full((1, D)),
]


def _tc_mid(h, accp, denp, g, be, wT, b, ai, aj, relp, wrT, br, ar):
    return pl.pallas_call(
        _tc_mid_body,
        grid=(NBLK,),
        in_specs=_AGG_SPECS + _WEIGHT_SPECS,
        out_specs=[_row_spec()] + _SCORE_OUT_SPECS,
        out_shape=[jax.ShapeDtypeStruct((NP, D), jnp.float32)] + _SCORE_OUT_SHAPES,
    )(h, accp, denp, g, be, wT, b, ai, aj, relp, wrT, br, ar)


def _tc_post(h, accp, denp, g, be):
    return pl.pallas_call(
        _tc_post_body,
        grid=(NBLK,),
        in_specs=_AGG_SPECS,
        out_specs=_row_spec(),
        out_shape=jax.ShapeDtypeStruct((NP, D), jnp.float32),
    )(h, accp, denp, g, be)


BB = 16
CHUNKS = EP // CH
NCH0 = 160
NCH1 = 0
CB1 = NS * NCH0


@functools.partial(
    pl.kernel,
    out_type=[
        jax.ShapeDtypeStruct((CHUNKS, CH), jnp.float32),
        jax.ShapeDtypeStruct((NT, NP), jnp.float32),
    ],
    mesh=plsc.VectorSubcoreMesh(core_axis_name="c", subcore_axis_name="s",
                                num_cores=NC, num_subcores=NS),
    compiler_params=pltpu.CompilerParams(needs_layout_passes=False),
    scratch_types=[
        pltpu.VMEM((BB, CH), jnp.int32),
        pltpu.VMEM((BB, CH), jnp.int32),
        pltpu.VMEM((BB, CH), jnp.int32),
        pltpu.VMEM((BB, CH), jnp.float32),
        pltpu.VMEM((NP,), jnp.float32),
        pltpu.VMEM((NP,), jnp.float32),
        pltpu.VMEM((RP,), jnp.float32),
        pltpu.VMEM((16,), jnp.float32),
        pltpu.VMEM((NP,), jnp.float32),
    ],
)
def _sc_weights(src_hbm, dst_hbm, typ_hbm, si_hbm, sj_hbm, sr_hbm, c_hbm,
                w_hbm, denp_hbm,
                srcb, dstb, typb, wb, siv, sjv, srv, cv, denv):
    cid = lax.axis_index("c")
    sid = lax.axis_index("s")
    wid = cid * NS + sid

    pltpu.sync_copy(si_hbm, siv)
    pltpu.sync_copy(sj_hbm, sjv)
    pltpu.sync_copy(sr_hbm, srv)
    pltpu.sync_copy(c_hbm, cv)

    zeros16 = jnp.zeros((16,), jnp.float32)

    def _zden(i, carry):
        denv[pl.ds(i * 16, 16)] = zeros16
        return carry
    lax.fori_loop(0, NP // 16, _zden, 0)

    cvec = cv[...]

    def _do(cbase, nch):
        for blk in range(nch // BB):
            bbase = cbase + blk * BB
            pltpu.sync_copy(src_hbm.at[pl.ds(bbase, BB)], srcb)
            pltpu.sync_copy(dst_hbm.at[pl.ds(bbase, BB)], dstb)
            pltpu.sync_copy(typ_hbm.at[pl.ds(bbase, BB)], typb)

            def _edge_scalar(r, carry):
                for q in range(8):
                    col = q * 16
                    d = dstb[r, pl.ds(col, 16)]
                    s = srcb[r, pl.ds(col, 16)]
                    t = typb[r, pl.ds(col, 16)]
                    gi = plsc.load_gather(siv, [d])
                    gj = plsc.load_gather(sjv, [s])
                    gr = plsc.load_gather(srv, [t])
                    tt = gi + gj + gr
                    a = jnp.where(tt > 0, tt, 0.2 * tt)
                    w = jnp.exp(a - cvec)
                    gid = (bbase + r) * CH + col + lax.iota(jnp.int32, 16)
                    w = jnp.where(gid < E, w, 0.0)
                    wb[r, pl.ds(col, 16)] = w
                    plsc.addupdate_scatter(denv, [d], w)
                return carry
            lax.fori_loop(0, BB, _edge_scalar, 0)
            pltpu.sync_copy(wb, w_hbm.at[pl.ds(bbase, BB)])

    @pl.when(cid == 0)
    def _():
        _do(sid * NCH0, NCH0)

    @pl.when(cid == 1)
    def _():
        _do(CB1 + sid * NCH1, NCH1)

    pltpu.sync_copy(denv, denp_hbm.at[wid])


@functools.partial(
    pl.kernel,
    out_type=jax.ShapeDtypeStruct((NC * NP, D), jnp.float32),
    mesh=plsc.VectorSubcoreMesh(core_axis_name="c", subcore_axis_name="s",
                                num_cores=NC, num_subcores=NS),
    compiler_params=pltpu.CompilerParams(needs_layout_passes=False),
    scratch_types=[
        pltpu.VMEM((BB, CH), jnp.int32),
        pltpu.VMEM((BB, CH), jnp.int32),
        pltpu.VMEM((BB, CH), jnp.float32),
        pltpu.VMEM((CH, D), jnp.float32),
        pltpu.VMEM((CH, D), jnp.float32),
        pltpu.VMEM_SHARED((NP, D), jnp.float32),
        pltpu.SemaphoreType.DMA,
        pltpu.SemaphoreType.DMA,
    ],
)
def _sc_rows(hx_hbm, src_hbm, dst_hbm, w_hbm, accp_hbm,
             srcb, dstb, wb, rows0, rows1, accs, sem0, sem1):
    cid = lax.axis_index("c")
    sid = lax.axis_index("s")
    wid = cid * NS + sid

    zeros16 = jnp.zeros((16,), jnp.float32)

    def _zrow(e, carry):
        for c in range(8):
            rows0[e, pl.ds(c * 16, 16)] = zeros16
        return carry
    lax.fori_loop(0, CH, _zrow, 0)

    row_base = sid * RPT
    for k in range(RPT // CH):
        pltpu.sync_copy(rows0, accs.at[pl.ds(row_base + k * CH, CH)])
    plsc.subcore_barrier()

    def _scale(rows, wrow):
        def _sc16(g, c2):
            wvec = wrow[pl.ds(g * 16, 16)]
            for l in range(16):
                ws = wvec[l]
                e = g * 16 + l
                for c in range(8):
                    sl = pl.ds(c * 16, 16)
                    rows[e, sl] = rows[e, sl] * ws
            return c2
        lax.fori_loop(0, CH // 16, _sc16, 0)

    def _do(cbase, nch):
        for blk in range(nch // BB):
            bbase = cbase + blk * BB
            pltpu.sync_copy(src_hbm.at[pl.ds(bbase, BB)], srcb)
            pltpu.sync_copy(dst_hbm.at[pl.ds(bbase, BB)], dstb)
            pltpu.sync_copy(w_hbm.at[pl.ds(bbase, BB)], wb)
            pltpu.async_copy(hx_hbm.at[srcb.at[0]], rows0, sem0)

            def _rowpair(jj, carry):
                j0 = 2 * jj
                pltpu.make_async_copy(hx_hbm.at[srcb.at[j0]], rows0, sem0).wait()
                pltpu.async_copy(hx_hbm.at[srcb.at[j0 + 1]], rows1, sem1)
                _scale(rows0, wb.at[j0])
                pltpu.sync_copy(rows0, accs.at[dstb.at[j0]], add=True)
                pltpu.make_async_copy(hx_hbm.at[srcb.at[j0 + 1]], rows1,
                                      sem1).wait()

                @pl.when(jj < BB // 2 - 1)
                def _():
                    pltpu.async_copy(hx_hbm.at[srcb.at[j0 + 2]], rows0, sem0)
                _scale(rows1, wb.at[j0 + 1])
                pltpu.sync_copy(rows1, accs.at[dstb.at[j0 + 1]], add=True)
                return carry
            lax.fori_loop(0, BB // 2, _rowpair, 0)

    @pl.when(cid == 0)
    def _():
        _do(sid * NCH0, NCH0)

    @pl.when(cid == 1)
    def _():
        _do(CB1 + sid * NCH1, NCH1)

    plsc.subcore_barrier()

    for k in range(RPT // CH):
        sl = pl.ds(row_base + k * CH, CH)
        pltpu.sync_copy(accs.at[sl], rows0)
        pltpu.sync_copy(rows0, accp_hbm.at[pl.ds(cid * NP + row_base + k * CH, CH)])


def _sc_agg(hx, src, dst, typ, si, sj, sr, c):
    w, denp = _sc_weights(src, dst, typ, si, sj, sr, c)
    accp = _sc_rows(hx, src, dst, w)
    return accp, denp


def _shift_const(bm, sr):
    bm3 = bm.reshape(NBLK, 8, D)
    c = jnp.max(bm3[:, 0, 0]) + jnp.max(bm3[:, 4, 0]) + jnp.max(sr)
    c = jnp.where(c > 0, c, 0.2 * c)
    return jnp.full((16,), c, jnp.float32)


def kernel(x, edge_index, edge_type, embedding,
           rel_emb1, W_lin1, b_lin1, W_rel1, b_rel1, attn1, gamma1, beta1,
           rel_emb2, W_lin2, b_lin2, W_rel2, b_rel2, attn2, gamma2, beta2):
    h = jnp.take(embedding, x, axis=0)
    h = jnp.pad(h, ((0, NP - N), (0, 0)))
    src = jnp.pad(edge_index[0], (0, EP - E)).reshape(CHUNKS, CH)
    dst = jnp.pad(edge_index[1], (0, EP - E)).reshape(CHUNKS, CH)
    typ = jnp.pad(edge_type, (0, EP - E)).reshape(CHUNKS, CH)
    relp1 = jnp.pad(rel_emb1, ((0, RP - R), (0, 0)))
    relp2 = jnp.pad(rel_emb2, ((0, RP - R), (0, 0)))

    def layer_weights(W_lin, b_lin, W_rel, b_rel, attn, relp):
        return (W_lin.T, b_lin[None], attn[:, :D], attn[:, D:2 * D],
                relp, W_rel.T, b_rel[None], attn[:, 2 * D:])

    w1 = layer_weights(W_lin1, b_lin1, W_rel1, b_rel1, attn1, relp1)
    w2 = layer_weights(W_lin2, b_lin2, W_rel2, b_rel2, attn2, relp2)

    hx1, si1, sj1, sr1, bm1 = _tc_pre(h, *w1)
    c1 = _shift_const(bm1, sr1)
    accp1, denp1 = _sc_agg(hx1, src, dst, typ,
                           si1.reshape(NP), sj1.reshape(NP),
                           sr1.reshape(RP), c1)

    h2, hx2, si2, sj2, sr2, bm2 = _tc_mid(
        h, accp1.reshape(NC, NP, D), denp1, gamma1[None], beta1[None], *w2)
    c2 = _shift_const(bm2, sr2)
    accp2, denp2 = _sc_agg(hx2, src, dst, typ,
                           si2.reshape(NP), sj2.reshape(NP),
                           sr2.reshape(RP), c2)

    hf = _tc_post(h2, accp2.reshape(NC, NP, D), denp2,
                  gamma2[None], beta2[None])
    return hf[:N]


# speedup vs baseline: 1.0398x; 1.0398x over previous
import functools

import jax
import jax.numpy as jnp
from jax import lax
from jax.experimental import pallas as pl
from jax.experimental.pallas import tpu as pltpu
from jax.experimental.pallas import tpu_sc as plsc

N = 10000
E = 320000
D = 128
R = 50

NP = 10240
EP = 327680
RP = 64
NC = 2
NS = 16
NT = NC * NS
ET = EP // NT
CH = 128
NCH = ET // CH
RPT = NP // NS
BLK = 512
NBLK = NP // BLK


def _proj_scores(i, h, wT_ref, b_ref, ai_ref, aj_ref, relp_ref, wrT_ref,
                 br_ref, ar_ref, hx_ref, hxb_ref, si_ref, sj_ref, sr_ref,
                 bm_ref):
    hx = jnp.dot(h, wT_ref[...], preferred_element_type=jnp.float32) + b_ref[...]
    hx_ref[...] = hx
    hxb_ref[...] = hx
    si = jnp.sum(hx * ai_ref[...], axis=1, keepdims=True)
    sj = jnp.sum(hx * aj_ref[...], axis=1, keepdims=True)
    si_ref[...] = si
    sj_ref[...] = sj
    bm_ref[...] = jnp.concatenate(
        [jnp.broadcast_to(jnp.max(si), (4, D)),
         jnp.broadcast_to(jnp.max(sj), (4, D))], axis=0)

    @pl.when(i == 0)
    def _():
        relp = jnp.dot(relp_ref[...], wrT_ref[...],
                       preferred_element_type=jnp.float32) + br_ref[...]
        sr_ref[...] = jnp.sum(relp * ar_ref[...], axis=1, keepdims=True)


def _tc_pre_body(h_ref, wT_ref, b_ref, ai_ref, aj_ref, relp_ref, wrT_ref,
                 br_ref, ar_ref, hx_ref, hxb_ref, si_ref, sj_ref, sr_ref,
                 bm_ref):
    i = pl.program_id(0)
    _proj_scores(i, h_ref[...], wT_ref, b_ref, ai_ref, aj_ref, relp_ref,
                 wrT_ref, br_ref, ar_ref, hx_ref, hxb_ref, si_ref, sj_ref,
                 sr_ref, bm_ref)


def _agg_norm(h_ref, acc_ref, den_ref, g_ref, be_ref):
    acc = acc_ref[0] + acc_ref[1]
    den = jnp.sum(den_ref[...], axis=0)
    out = acc / (den[:, None] + 1e-16)
    hsum = h_ref[...] + out
    mean = jnp.mean(hsum, axis=1, keepdims=True)
    var = jnp.mean((hsum - mean) ** 2, axis=1, keepdims=True)
    hn = (hsum - mean) * lax.rsqrt(var + 1e-5) * g_ref[...] + be_ref[...]
    return jnp.maximum(hn, 0.0)


def _tc_mid_body(h_ref, acc_ref, den_ref, g_ref, be_ref, wT_ref, b_ref,
                 ai_ref, aj_ref, relp_ref, wrT_ref, br_ref, ar_ref,
                 h2_ref, hx_ref, hxb_ref, si_ref, sj_ref, sr_ref, bm_ref):
    i = pl.program_id(0)
    h2 = _agg_norm(h_ref, acc_ref, den_ref, g_ref, be_ref)
    h2_ref[...] = h2
    _proj_scores(i, h2, wT_ref, b_ref, ai_ref, aj_ref, relp_ref, wrT_ref,
                 br_ref, ar_ref, hx_ref, hxb_ref, si_ref, sj_ref, sr_ref,
                 bm_ref)


def _tc_post_body(h_ref, acc_ref, den_ref, g_ref, be_ref, out_ref):
    out_ref[...] = _agg_norm(h_ref, acc_ref, den_ref, g_ref, be_ref)


def _row_spec():
    return pl.BlockSpec((BLK, D), lambda i: (i, 0))


def _full(shape):
    nd = len(shape)
    return pl.BlockSpec(shape, lambda i: (0,) * nd)


_SCORE_OUT_SHAPES = [
    jax.ShapeDtypeStruct((NP, D), jnp.float32),
    jax.ShapeDtypeStruct((NP, D), jnp.float32),
    jax.ShapeDtypeStruct((NP, 1), jnp.float32),
    jax.ShapeDtypeStruct((NP, 1), jnp.float32),
    jax.ShapeDtypeStruct((RP, 1), jnp.float32),
    jax.ShapeDtypeStruct((NBLK * 8, D), jnp.float32),
]
_SCORE_OUT_SPECS = [
    _row_spec(),
    _row_spec(),
    pl.BlockSpec((BLK, 1), lambda i: (i, 0)),
    pl.BlockSpec((BLK, 1), lambda i: (i, 0)),
    _full((RP, 1)),
    pl.BlockSpec((8, D), lambda i: (i, 0)),
]
_WEIGHT_SPECS = [
    _full((D, D)),
    _full((1, D)),
    _full((1, D)),
    _full((1, D)),
    _full((RP, D)),
    _full((D, D)),
    _full((1, D)),
    _full((1, D)),
]


def _tc_pre(h, wT, b, ai, aj, relp, wrT, br, ar):
    return pl.pallas_call(
        _tc_pre_body,
        grid=(NBLK,),
        in_specs=[_row_spec()] + _WEIGHT_SPECS,
        out_specs=_SCORE_OUT_SPECS,
        out_shape=_SCORE_OUT_SHAPES,
    )(h, wT, b, ai, aj, relp, wrT, br, ar)


_AGG_SPECS = [
    _row_spec(),
    pl.BlockSpec((NC, BLK, D), lambda i: (0, i, 0)),
    pl.BlockSpec((NT, BLK), lambda i: (0, i)),
    _full((1, D)),
    _full((1, D)),
]


def _tc_mid(h, accp, denp, g, be, wT, b, ai, aj, relp, wrT, br, ar):
    return pl.pallas_call(
        _tc_mid_body,
        grid=(NBLK,),
        in_specs=_AGG_SPECS + _WEIGHT_SPECS,
        out_specs=[_row_spec()] + _SCORE_OUT_SPECS,
        out_shape=[jax.ShapeDtypeStruct((NP, D), jnp.float32)] + _SCORE_OUT_SHAPES,
    )(h, accp, denp, g, be, wT, b, ai, aj, relp, wrT, br, ar)


def _tc_post(h, accp, denp, g, be):
    return pl.pallas_call(
        _tc_post_body,
        grid=(NBLK,),
        in_specs=_AGG_SPECS,
        out_specs=_row_spec(),
        out_shape=jax.ShapeDtypeStruct((NP, D), jnp.float32),
    )(h, accp, denp, g, be)


BB = 16
CHUNKS = EP // CH
NCH0 = 80
NCH1 = 80
CB1 = NS * NCH0


@functools.partial(
    pl.kernel,
    out_type=[
        jax.ShapeDtypeStruct((CHUNKS, CH), jnp.float32),
        jax.ShapeDtypeStruct((NT, NP), jnp.float32),
    ],
    mesh=plsc.VectorSubcoreMesh(core_axis_name="c", subcore_axis_name="s",
                                num_cores=NC, num_subcores=NS),
    compiler_params=pltpu.CompilerParams(needs_layout_passes=False),
    scratch_types=[
        pltpu.VMEM((BB, CH), jnp.int32),
        pltpu.VMEM((BB, CH), jnp.int32),
        pltpu.VMEM((BB, CH), jnp.int32),
        pltpu.VMEM((BB, CH), jnp.float32),
        pltpu.VMEM((NP,), jnp.float32),
        pltpu.VMEM((NP,), jnp.float32),
        pltpu.VMEM((RP,), jnp.float32),
        pltpu.VMEM((16,), jnp.float32),
        pltpu.VMEM((NP,), jnp.float32),
    ],
)
def _sc_weights(src_hbm, dst_hbm, typ_hbm, si_hbm, sj_hbm, sr_hbm, c_hbm,
                w_hbm, denp_hbm,
                srcb, dstb, typb, wb, siv, sjv, srv, cv, denv):
    cid = lax.axis_index("c")
    sid = lax.axis_index("s")
    wid = cid * NS + sid

    pltpu.sync_copy(si_hbm, siv)
    pltpu.sync_copy(sj_hbm, sjv)
    pltpu.sync_copy(sr_hbm, srv)
    pltpu.sync_copy(c_hbm, cv)

    zeros16 = jnp.zeros((16,), jnp.float32)

    def _zden(i, carry):
        denv[pl.ds(i * 16, 16)] = zeros16
        return carry
    lax.fori_loop(0, NP // 16, _zden, 0)

    cvec = cv[...]

    def _do(cbase, nch):
        for blk in range(nch // BB):
            bbase = cbase + blk * BB
            pltpu.sync_copy(src_hbm.at[pl.ds(bbase, BB)], srcb)
            pltpu.sync_copy(dst_hbm.at[pl.ds(bbase, BB)], dstb)
            pltpu.sync_copy(typ_hbm.at[pl.ds(bbase, BB)], typb)

            def _edge_scalar(r, carry):
                for q in range(8):
                    col = q * 16
                    d = dstb[r, pl.ds(col, 16)]
                    s = srcb[r, pl.ds(col, 16)]
                    t = typb[r, pl.ds(col, 16)]
                    gi = plsc.load_gather(siv, [d])
                    gj = plsc.load_gather(sjv, [s])
                    gr = plsc.load_gather(srv, [t])
                    tt = gi + gj + gr
                    a = jnp.where(tt > 0, tt, 0.2 * tt)
                    w = jnp.exp(a - cvec)
                    gid = (bbase + r) * CH + col + lax.iota(jnp.int32, 16)
                    w = jnp.where(gid < E, w, 0.0)
                    wb[r, pl.ds(col, 16)] = w
                    plsc.addupdate_scatter(denv, [d], w)
                return carry
            lax.fori_loop(0, BB, _edge_scalar, 0)
            pltpu.sync_copy(wb, w_hbm.at[pl.ds(bbase, BB)])

    @pl.when(cid == 0)
    def _():
        _do(sid * NCH0, NCH0)

    @pl.when(cid == 1)
    def _():
        _do(CB1 + sid * NCH1, NCH1)

    pltpu.sync_copy(denv, denp_hbm.at[wid])


@functools.partial(
    pl.kernel,
    out_type=jax.ShapeDtypeStruct((NC * NP, D), jnp.float32),
    mesh=plsc.VectorSubcoreMesh(core_axis_name="c", subcore_axis_name="s",
                                num_cores=NC, num_subcores=NS),
    compiler_params=pltpu.CompilerParams(needs_layout_passes=False),
    scratch_types=[
        pltpu.VMEM((BB, CH), jnp.int32),
        pltpu.VMEM((BB, CH), jnp.int32),
        pltpu.VMEM((BB, CH), jnp.float32),
        pltpu.VMEM((CH, D), jnp.float32),
        pltpu.VMEM((CH, D), jnp.float32),
        pltpu.VMEM_SHARED((NP, D), jnp.float32),
        pltpu.SemaphoreType.DMA,
        pltpu.SemaphoreType.DMA,
    ],
)
def _sc_rows(hx0_hbm, hx1_hbm, src_hbm, dst_hbm, w_hbm, accp_hbm,
             srcb, dstb, wb, rows0, rows1, accs, sem0, sem1):
    cid = lax.axis_index("c")
    sid = lax.axis_index("s")
    wid = cid * NS + sid

    zeros16 = jnp.zeros((16,), jnp.float32)

    def _zrow(e, carry):
        for c in range(8):
            rows0[e, pl.ds(c * 16, 16)] = zeros16
        return carry
    lax.fori_loop(0, CH, _zrow, 0)

    row_base = sid * RPT
    for k in range(RPT // CH):
        pltpu.sync_copy(rows0, accs.at[pl.ds(row_base + k * CH, CH)])
    plsc.subcore_barrier()

    def _scale(rows, wrow):
        def _sc16(g, c2):
            wvec = wrow[pl.ds(g * 16, 16)]
            for l in range(16):
                ws = wvec[l]
                e = g * 16 + l
                for c in range(8):
                    sl = pl.ds(c * 16, 16)
                    rows[e, sl] = rows[e, sl] * ws
            return c2
        lax.fori_loop(0, CH // 16, _sc16, 0)

    def _do(hx_hbm, cbase, nch):
        for blk in range(nch // BB):
            bbase = cbase + blk * BB
            pltpu.sync_copy(src_hbm.at[pl.ds(bbase, BB)], srcb)
            pltpu.sync_copy(dst_hbm.at[pl.ds(bbase, BB)], dstb)
            pltpu.sync_copy(w_hbm.at[pl.ds(bbase, BB)], wb)
            pltpu.async_copy(hx_hbm.at[srcb.at[0]], rows0, sem0)

            def _rowpair(jj, carry):
                j0 = 2 * jj
                pltpu.make_async_copy(hx_hbm.at[srcb.at[j0]], rows0, sem0).wait()
                pltpu.async_copy(hx_hbm.at[srcb.at[j0 + 1]], rows1, sem1)
                _scale(rows0, wb.at[j0])
                pltpu.sync_copy(rows0, accs.at[dstb.at[j0]], add=True)
                pltpu.make_async_copy(hx_hbm.at[srcb.at[j0 + 1]], rows1,
                                      sem1).wait()

                @pl.when(jj < BB // 2 - 1)
                def _():
                    pltpu.async_copy(hx_hbm.at[srcb.at[j0 + 2]], rows0, sem0)
                _scale(rows1, wb.at[j0 + 1])
                pltpu.sync_copy(rows1, accs.at[dstb.at[j0 + 1]], add=True)
                return carry
            lax.fori_loop(0, BB // 2, _rowpair, 0)

    @pl.when(cid == 0)
    def _():
        _do(hx0_hbm, sid * NCH0, NCH0)

    @pl.when(cid == 1)
    def _():
        _do(hx1_hbm, CB1 + sid * NCH1, NCH1)

    plsc.subcore_barrier()

    for k in range(RPT // CH):
        sl = pl.ds(row_base + k * CH, CH)
        pltpu.sync_copy(accs.at[sl], rows0)
        pltpu.sync_copy(rows0, accp_hbm.at[pl.ds(cid * NP + row_base + k * CH, CH)])


def _sc_agg(hx, hxb, src, dst, typ, si, sj, sr, c):
    w, denp = _sc_weights(src, dst, typ, si, sj, sr, c)
    accp = _sc_rows(hx, hxb, src, dst, w)
    return accp, denp


def _shift_const(bm, sr):
    bm3 = bm.reshape(NBLK, 8, D)
    c = jnp.max(bm3[:, 0, 0]) + jnp.max(bm3[:, 4, 0]) + jnp.max(sr)
    c = jnp.where(c > 0, c, 0.2 * c)
    return jnp.full((16,), c, jnp.float32)


def kernel(x, edge_index, edge_type, embedding,
           rel_emb1, W_lin1, b_lin1, W_rel1, b_rel1, attn1, gamma1, beta1,
           rel_emb2, W_lin2, b_lin2, W_rel2, b_rel2, attn2, gamma2, beta2):
    h = jnp.take(embedding, x, axis=0)
    h = jnp.pad(h, ((0, NP - N), (0, 0)))
    src = jnp.pad(edge_index[0], (0, EP - E)).reshape(CHUNKS, CH)
    dst = jnp.pad(edge_index[1], (0, EP - E)).reshape(CHUNKS, CH)
    typ = jnp.pad(edge_type, (0, EP - E)).reshape(CHUNKS, CH)
    relp1 = jnp.pad(rel_emb1, ((0, RP - R), (0, 0)))
    relp2 = jnp.pad(rel_emb2, ((0, RP - R), (0, 0)))

    def layer_weights(W_lin, b_lin, W_rel, b_rel, attn, relp):
        return (W_lin.T, b_lin[None], attn[:, :D], attn[:, D:2 * D],
                relp, W_rel.T, b_rel[None], attn[:, 2 * D:])

    w1 = layer_weights(W_lin1, b_lin1, W_rel1, b_rel1, attn1, relp1)
    w2 = layer_weights(W_lin2, b_lin2, W_rel2, b_rel2, attn2, relp2)

    hx1, hxb1, si1, sj1, sr1, bm1 = _tc_pre(h, *w1)
    c1 = _shift_const(bm1, sr1)
    accp1, denp1 = _sc_agg(hx1, hxb1, src, dst, typ,
                           si1.reshape(NP), sj1.reshape(NP),
                           sr1.reshape(RP), c1)

    h2, hx2, hxb2, si2, sj2, sr2, bm2 = _tc_mid(
        h, accp1.reshape(NC, NP, D), denp1, gamma1[None], beta1[None], *w2)
    c2 = _shift_const(bm2, sr2)
    accp2, denp2 = _sc_agg(hx2, hxb2, src, dst, typ,
                           si2.reshape(NP), sj2.reshape(NP),
                           sr2.reshape(RP), c2)

    hf = _tc_post(h2, accp2.reshape(NC, NP, D), denp2,
                  gamma2[None], beta2[None])
    return hf[:N]


# speedup vs baseline: 1.1951x; 1.1493x over previous
import functools

import jax
import jax.numpy as jnp
from jax import lax
from jax.experimental import pallas as pl
from jax.experimental.pallas import tpu as pltpu
from jax.experimental.pallas import tpu_sc as plsc

N = 10000
E = 320000
D = 128
R = 50

NP = 10240
EP = 327680
RP = 64
NC = 2
NS = 16
NT = NC * NS
ET = EP // NT
CH = 128
NCH = ET // CH
RPT = NP // NS
BLK = 512
NBLK = NP // BLK


def _proj_scores(i, h, wT_ref, b_ref, ai_ref, aj_ref, relp_ref, wrT_ref,
                 br_ref, ar_ref, hx_ref, si_ref, sj_ref, sr_ref, bm_ref):
    hx = jnp.dot(h, wT_ref[...], preferred_element_type=jnp.float32) + b_ref[...]
    hx_ref[...] = hx
    si = jnp.sum(hx * ai_ref[...], axis=1, keepdims=True)
    sj = jnp.sum(hx * aj_ref[...], axis=1, keepdims=True)
    si_ref[...] = si
    sj_ref[...] = sj
    bm_ref[...] = jnp.concatenate(
        [jnp.broadcast_to(jnp.max(si), (4, D)),
         jnp.broadcast_to(jnp.max(sj), (4, D))], axis=0)

    @pl.when(i == 0)
    def _():
        relp = jnp.dot(relp_ref[...], wrT_ref[...],
                       preferred_element_type=jnp.float32) + br_ref[...]
        sr_ref[...] = jnp.sum(relp * ar_ref[...], axis=1, keepdims=True)


def _tc_pre_body(h_ref, wT_ref, b_ref, ai_ref, aj_ref, relp_ref, wrT_ref,
                 br_ref, ar_ref, hx_ref, si_ref, sj_ref, sr_ref, bm_ref):
    i = pl.program_id(0)
    _proj_scores(i, h_ref[...], wT_ref, b_ref, ai_ref, aj_ref, relp_ref,
                 wrT_ref, br_ref, ar_ref, hx_ref, si_ref, sj_ref, sr_ref,
                 bm_ref)


def _agg_norm(h_ref, acc_ref, den_ref, g_ref, be_ref):
    acc = acc_ref[0] + acc_ref[1]
    den = jnp.sum(den_ref[...], axis=0)
    out = acc / (den[:, None] + 1e-16)
    hsum = h_ref[...] + out
    mean = jnp.mean(hsum, axis=1, keepdims=True)
    var = jnp.mean((hsum - mean) ** 2, axis=1, keepdims=True)
    hn = (hsum - mean) * lax.rsqrt(var + 1e-5) * g_ref[...] + be_ref[...]
    return jnp.maximum(hn, 0.0)


def _tc_mid_body(h_ref, acc_ref, den_ref, g_ref, be_ref, wT_ref, b_ref,
                 ai_ref, aj_ref, relp_ref, wrT_ref, br_ref, ar_ref,
                 h2_ref, hx_ref, si_ref, sj_ref, sr_ref, bm_ref):
    i = pl.program_id(0)
    h2 = _agg_norm(h_ref, acc_ref, den_ref, g_ref, be_ref)
    h2_ref[...] = h2
    _proj_scores(i, h2, wT_ref, b_ref, ai_ref, aj_ref, relp_ref, wrT_ref,
                 br_ref, ar_ref, hx_ref, si_ref, sj_ref, sr_ref, bm_ref)


def _tc_post_body(h_ref, acc_ref, den_ref, g_ref, be_ref, out_ref):
    out_ref[...] = _agg_norm(h_ref, acc_ref, den_ref, g_ref, be_ref)


def _row_spec():
    return pl.BlockSpec((BLK, D), lambda i: (i, 0))


def _full(shape):
    nd = len(shape)
    return pl.BlockSpec(shape, lambda i: (0,) * nd)


_SCORE_OUT_SHAPES = [
    jax.ShapeDtypeStruct((NP, D), jnp.float32),
    jax.ShapeDtypeStruct((NP, 1), jnp.float32),
    jax.ShapeDtypeStruct((NP, 1), jnp.float32),
    jax.ShapeDtypeStruct((RP, 1), jnp.float32),
    jax.ShapeDtypeStruct((NBLK * 8, D), jnp.float32),
]
_SCORE_OUT_SPECS = [
    _row_spec(),
    pl.BlockSpec((BLK, 1), lambda i: (i, 0)),
    pl.BlockSpec((BLK, 1), lambda i: (i, 0)),
    _full((RP, 1)),
    pl.BlockSpec((8, D), lambda i: (i, 0)),
]
_WEIGHT_SPECS = [
    _full((D, D)),
    _full((1, D)),
    _full((1, D)),
    _full((1, D)),
    _full((RP, D)),
    _full((D, D)),
    _full((1, D)),
    _full((1, D)),
]


def _tc_pre(h, wT, b, ai, aj, relp, wrT, br, ar):
    return pl.pallas_call(
        _tc_pre_body,
        grid=(NBLK,),
        in_specs=[_row_spec()] + _WEIGHT_SPECS,
        out_specs=_SCORE_OUT_SPECS,
        out_shape=_SCORE_OUT_SHAPES,
    )(h, wT, b, ai, aj, relp, wrT, br, ar)


_AGG_SPECS = [
    _row_spec(),
    pl.BlockSpec((NC, BLK, D), lambda i: (0, i, 0)),
    pl.BlockSpec((NT, BLK), lambda i: (0, i)),
    _full((1, D)),
    _full((1, D)),
]


def _tc_mid(h, accp, denp, g, be, wT, b, ai, aj, relp, wrT, br, ar):
    return pl.pallas_call(
        _tc_mid_body,
        grid=(NBLK,),
        in_specs=_AGG_SPECS + _WEIGHT_SPECS,
        out_specs=[_row_spec()] + _SCORE_OUT_SPECS,
        out_shape=[jax.ShapeDtypeStruct((NP, D), jnp.float32)] + _SCORE_OUT_SHAPES,
    )(h, accp, denp, g, be, wT, b, ai, aj, relp, wrT, br, ar)


def _tc_post(h, accp, denp, g, be):
    return pl.pallas_call(
        _tc_post_body,
        grid=(NBLK,),
        in_specs=_AGG_SPECS,
        out_specs=_row_spec(),
        out_shape=jax.ShapeDtypeStruct((NP, D), jnp.float32),
    )(h, accp, denp, g, be)


BB = 16
CHUNKS = EP // CH
NCH0 = 80
NCH1 = 80
CB1 = NS * NCH0


@functools.partial(
    pl.kernel,
    out_type=[
        jax.ShapeDtypeStruct((CHUNKS, CH), jnp.float32),
        jax.ShapeDtypeStruct((NT, NP), jnp.float32),
    ],
    mesh=plsc.VectorSubcoreMesh(core_axis_name="c", subcore_axis_name="s",
                                num_cores=NC, num_subcores=NS),
    compiler_params=pltpu.CompilerParams(needs_layout_passes=False),
    scratch_types=[
        pltpu.VMEM((BB, CH), jnp.int32),
        pltpu.VMEM((BB, CH), jnp.int32),
        pltpu.VMEM((BB, CH), jnp.int32),
        pltpu.VMEM((BB, CH), jnp.float32),
        pltpu.VMEM((NP,), jnp.float32),
        pltpu.VMEM((NP,), jnp.float32),
        pltpu.VMEM((RP,), jnp.float32),
        pltpu.VMEM((16,), jnp.float32),
        pltpu.VMEM((NP,), jnp.float32),
    ],
)
def _sc_weights(src_hbm, dst_hbm, typ_hbm, si_hbm, sj_hbm, sr_hbm, c_hbm,
                w_hbm, denp_hbm,
                srcb, dstb, typb, wb, siv, sjv, srv, cv, denv):
    cid = lax.axis_index("c")
    sid = lax.axis_index("s")
    wid = cid * NS + sid

    pltpu.sync_copy(si_hbm, siv)
    pltpu.sync_copy(sj_hbm, sjv)
    pltpu.sync_copy(sr_hbm, srv)
    pltpu.sync_copy(c_hbm, cv)

    zeros16 = jnp.zeros((16,), jnp.float32)

    def _zden(i, carry):
        denv[pl.ds(i * 16, 16)] = zeros16
        return carry
    lax.fori_loop(0, NP // 16, _zden, 0)

    cvec = cv[...]

    def _do(cbase, nch):
        for blk in range(nch // BB):
            bbase = cbase + blk * BB
            pltpu.sync_copy(src_hbm.at[pl.ds(bbase, BB)], srcb)
            pltpu.sync_copy(dst_hbm.at[pl.ds(bbase, BB)], dstb)
            pltpu.sync_copy(typ_hbm.at[pl.ds(bbase, BB)], typb)

            def _edge_scalar(r, carry):
                for q in range(8):
                    col = q * 16
                    d = dstb[r, pl.ds(col, 16)]
                    s = srcb[r, pl.ds(col, 16)]
                    t = typb[r, pl.ds(col, 16)]
                    gi = plsc.load_gather(siv, [d])
                    gj = plsc.load_gather(sjv, [s])
                    gr = plsc.load_gather(srv, [t])
                    tt = gi + gj + gr
                    a = jnp.where(tt > 0, tt, 0.2 * tt)
                    w = jnp.exp(a - cvec)
                    gid = (bbase + r) * CH + col + lax.iota(jnp.int32, 16)
                    w = jnp.where(gid < E, w, 0.0)
                    wb[r, pl.ds(col, 16)] = w
                    plsc.addupdate_scatter(denv, [d], w)
                return carry
            lax.fori_loop(0, BB, _edge_scalar, 0)
            pltpu.sync_copy(wb, w_hbm.at[pl.ds(bbase, BB)])

    @pl.when(cid == 0)
    def _():
        _do(sid * NCH0, NCH0)

    @pl.when(cid == 1)
    def _():
        _do(CB1 + sid * NCH1, NCH1)

    pltpu.sync_copy(denv, denp_hbm.at[wid])


@functools.partial(
    pl.kernel,
    out_type=jax.ShapeDtypeStruct((NC * NP, D), jnp.float32),
    mesh=plsc.VectorSubcoreMesh(core_axis_name="c", subcore_axis_name="s",
                                num_cores=NC, num_subcores=NS),
    compiler_params=pltpu.CompilerParams(needs_layout_passes=False),
    scratch_types=[
        pltpu.VMEM((BB, CH), jnp.int32),
        pltpu.VMEM((BB, CH), jnp.int32),
        pltpu.VMEM((BB, CH), jnp.float32),
        pltpu.VMEM((CH, D), jnp.float32),
        pltpu.VMEM((CH, D), jnp.float32),
        pltpu.VMEM_SHARED((NP, D), jnp.float32),
        pltpu.SemaphoreType.DMA,
        pltpu.SemaphoreType.DMA,
        pltpu.SemaphoreType.DMA,
        pltpu.SemaphoreType.DMA,
    ],
)
def _sc_rows(hx_hbm, src_hbm, dst_hbm, w_hbm, accp_hbm,
             srcb, dstb, wb, rows0, rows1, accs, sem0, sem1, semA, semB):
    cid = lax.axis_index("c")
    sid = lax.axis_index("s")
    wid = cid * NS + sid

    zeros16 = jnp.zeros((16,), jnp.float32)

    def _zrow(e, carry):
        for c in range(8):
            rows0[e, pl.ds(c * 16, 16)] = zeros16
        return carry
    lax.fori_loop(0, CH, _zrow, 0)

    row_base = sid * RPT
    for k in range(RPT // CH):
        pltpu.sync_copy(rows0, accs.at[pl.ds(row_base + k * CH, CH)])
    plsc.subcore_barrier()

    def _scale(rows, wrow):
        def _sc16(g, c2):
            wvec = wrow[pl.ds(g * 16, 16)]
            for l in range(16):
                ws = wvec[l]
                e = g * 16 + l
                for c in range(8):
                    sl = pl.ds(c * 16, 16)
                    rows[e, sl] = rows[e, sl] * ws
            return c2
        lax.fori_loop(0, CH // 16, _sc16, 0)

    def _do(cbase, nch):
        for blk in range(nch // BB):
            bbase = cbase + blk * BB
            pltpu.sync_copy(src_hbm.at[pl.ds(bbase, BB)], srcb)
            pltpu.sync_copy(dst_hbm.at[pl.ds(bbase, BB)], dstb)
            pltpu.sync_copy(w_hbm.at[pl.ds(bbase, BB)], wb)
            pltpu.async_copy(hx_hbm.at[srcb.at[0]], rows0, sem0)

            def _rowpair(jj, carry):
                j0 = 2 * jj
                pltpu.make_async_copy(hx_hbm.at[srcb.at[j0]], rows0, sem0).wait()

                @pl.when(jj > 0)
                def _():
                    pltpu.make_async_copy(rows1, accs.at[dstb.at[j0]],
                                          semB).wait()
                pltpu.async_copy(hx_hbm.at[srcb.at[j0 + 1]], rows1, sem1)
                _scale(rows0, wb.at[j0])
                pltpu.async_copy(rows0, accs.at[dstb.at[j0]], semA, add=True)
                pltpu.make_async_copy(hx_hbm.at[srcb.at[j0 + 1]], rows1,
                                      sem1).wait()
                _scale(rows1, wb.at[j0 + 1])
                pltpu.make_async_copy(rows0, accs.at[dstb.at[j0]], semA).wait()

                @pl.when(jj < BB // 2 - 1)
                def _():
                    pltpu.async_copy(hx_hbm.at[srcb.at[j0 + 2]], rows0, sem0)
                pltpu.async_copy(rows1, accs.at[dstb.at[j0 + 1]], semB,
                                 add=True)
                return carry
            lax.fori_loop(0, BB // 2, _rowpair, 0)
            pltpu.make_async_copy(rows1, accs.at[dstb.at[0]], semB).wait()

    @pl.when(cid == 0)
    def _():
        _do(sid * NCH0, NCH0)

    @pl.when(cid == 1)
    def _():
        _do(CB1 + sid * NCH1, NCH1)

    plsc.subcore_barrier()

    for k in range(RPT // CH):
        sl = pl.ds(row_base + k * CH, CH)
        pltpu.sync_copy(accs.at[sl], rows0)
        pltpu.sync_copy(rows0, accp_hbm.at[pl.ds(cid * NP + row_base + k * CH, CH)])


def _sc_agg(hx, src, dst, typ, si, sj, sr, c):
    w, denp = _sc_weights(src, dst, typ, si, sj, sr, c)
    accp = _sc_rows(hx, src, dst, w)
    return accp, denp


def _shift_const(bm, sr):
    bm3 = bm.reshape(NBLK, 8, D)
    c = jnp.max(bm3[:, 0, 0]) + jnp.max(bm3[:, 4, 0]) + jnp.max(sr)
    c = jnp.where(c > 0, c, 0.2 * c)
    return jnp.full((16,), c, jnp.float32)


def kernel(x, edge_index, edge_type, embedding,
           rel_emb1, W_lin1, b_lin1, W_rel1, b_rel1, attn1, gamma1, beta1,
           rel_emb2, W_lin2, b_lin2, W_rel2, b_rel2, attn2, gamma2, beta2):
    h = jnp.take(embedding, x, axis=0)
    h = jnp.pad(h, ((0, NP - N), (0, 0)))
    src = jnp.pad(edge_index[0], (0, EP - E)).reshape(CHUNKS, CH)
    dst = jnp.pad(edge_index[1], (0, EP - E)).reshape(CHUNKS, CH)
    typ = jnp.pad(edge_type, (0, EP - E)).reshape(CHUNKS, CH)
    relp1 = jnp.pad(rel_emb1, ((0, RP - R), (0, 0)))
    relp2 = jnp.pad(rel_emb2, ((0, RP - R), (0, 0)))

    def layer_weights(W_lin, b_lin, W_rel, b_rel, attn, relp):
        return (W_lin.T, b_lin[None], attn[:, :D], attn[:, D:2 * D],
                relp, W_rel.T, b_rel[None], attn[:, 2 * D:])

    w1 = layer_weights(W_lin1, b_lin1, W_rel1, b_rel1, attn1, relp1)
    w2 = layer_weights(W_lin2, b_lin2, W_rel2, b_rel2, attn2, relp2)

    hx1, si1, sj1, sr1, bm1 = _tc_pre(h, *w1)
    c1 = _shift_const(bm1, sr1)
    accp1, denp1 = _sc_agg(hx1, src, dst, typ,
                           si1.reshape(NP), sj1.reshape(NP),
                           sr1.reshape(RP), c1)

    h2, hx2, si2, sj2, sr2, bm2 = _tc_mid(
        h, accp1.reshape(NC, NP, D), denp1, gamma1[None], beta1[None], *w2)
    c2 = _shift_const(bm2, sr2)
    accp2, denp2 = _sc_agg(hx2, src, dst, typ,
                           si2.reshape(NP), sj2.reshape(NP),
                           sr2.reshape(RP), c2)

    hf = _tc_post(h2, accp2.reshape(NC, NP, D), denp2,
                  gamma2[None], beta2[None])
    return hf[:N]


# speedup vs baseline: 1.3058x; 1.0927x over previous
import functools

import jax
import jax.numpy as jnp
from jax import lax
from jax.experimental import pallas as pl
from jax.experimental.pallas import tpu as pltpu
from jax.experimental.pallas import tpu_sc as plsc

N = 10000
E = 320000
D = 128
R = 50

NP = 10240
EP = 327680
RP = 64
NC = 2
NS = 16
NT = NC * NS
ET = EP // NT
CH = 128
NCH = ET // CH
RPT = NP // NS
BLK = 512
NBLK = NP // BLK


def _proj_scores(i, h, wT_ref, b_ref, ai_ref, aj_ref, relp_ref, wrT_ref,
                 br_ref, ar_ref, hx_ref, si_ref, sj_ref, sr_ref, bm_ref):
    hx = jnp.dot(h, wT_ref[...], preferred_element_type=jnp.float32) + b_ref[...]
    hx_ref[...] = hx
    si = jnp.sum(hx * ai_ref[...], axis=1, keepdims=True)
    sj = jnp.sum(hx * aj_ref[...], axis=1, keepdims=True)
    si_ref[...] = si
    sj_ref[...] = sj
    bm_ref[...] = jnp.concatenate(
        [jnp.broadcast_to(jnp.max(si), (4, D)),
         jnp.broadcast_to(jnp.max(sj), (4, D))], axis=0)

    @pl.when(i == 0)
    def _():
        relp = jnp.dot(relp_ref[...], wrT_ref[...],
                       preferred_element_type=jnp.float32) + br_ref[...]
        sr_ref[...] = jnp.sum(relp * ar_ref[...], axis=1, keepdims=True)


def _tc_pre_body(h_ref, wT_ref, b_ref, ai_ref, aj_ref, relp_ref, wrT_ref,
                 br_ref, ar_ref, hx_ref, si_ref, sj_ref, sr_ref, bm_ref):
    i = pl.program_id(0)
    _proj_scores(i, h_ref[...], wT_ref, b_ref, ai_ref, aj_ref, relp_ref,
                 wrT_ref, br_ref, ar_ref, hx_ref, si_ref, sj_ref, sr_ref,
                 bm_ref)


def _agg_norm(h_ref, acc_ref, den_ref, g_ref, be_ref):
    acc = acc_ref[0] + acc_ref[1]
    den = jnp.sum(den_ref[...], axis=0)
    out = acc / (den[:, None] + 1e-16)
    hsum = h_ref[...] + out
    mean = jnp.mean(hsum, axis=1, keepdims=True)
    var = jnp.mean((hsum - mean) ** 2, axis=1, keepdims=True)
    hn = (hsum - mean) * lax.rsqrt(var + 1e-5) * g_ref[...] + be_ref[...]
    return jnp.maximum(hn, 0.0)


def _tc_mid_body(h_ref, acc_ref, den_ref, g_ref, be_ref, wT_ref, b_ref,
                 ai_ref, aj_ref, relp_ref, wrT_ref, br_ref, ar_ref,
                 h2_ref, hx_ref, si_ref, sj_ref, sr_ref, bm_ref):
    i = pl.program_id(0)
    h2 = _agg_norm(h_ref, acc_ref, den_ref, g_ref, be_ref)
    h2_ref[...] = h2
    _proj_scores(i, h2, wT_ref, b_ref, ai_ref, aj_ref, relp_ref, wrT_ref,
                 br_ref, ar_ref, hx_ref, si_ref, sj_ref, sr_ref, bm_ref)


def _tc_post_body(h_ref, acc_ref, den_ref, g_ref, be_ref, out_ref):
    out_ref[...] = _agg_norm(h_ref, acc_ref, den_ref, g_ref, be_ref)


def _row_spec():
    return pl.BlockSpec((BLK, D), lambda i: (i, 0))


def _full(shape):
    nd = len(shape)
    return pl.BlockSpec(shape, lambda i: (0,) * nd)


_SCORE_OUT_SHAPES = [
    jax.ShapeDtypeStruct((NP, D), jnp.float32),
    jax.ShapeDtypeStruct((NP, 1), jnp.float32),
    jax.ShapeDtypeStruct((NP, 1), jnp.float32),
    jax.ShapeDtypeStruct((RP, 1), jnp.float32),
    jax.ShapeDtypeStruct((NBLK * 8, D), jnp.float32),
]
_SCORE_OUT_SPECS = [
    _row_spec(),
    pl.BlockSpec((BLK, 1), lambda i: (i, 0)),
    pl.BlockSpec((BLK, 1), lambda i: (i, 0)),
    _full((RP, 1)),
    pl.BlockSpec((8, D), lambda i: (i, 0)),
]
_WEIGHT_SPECS = [
    _full((D, D)),
    _full((1, D)),
    _full((1, D)),
    _full((1, D)),
    _full((RP, D)),
    _full((D, D)),
    _full((1, D)),
    _full((1, D)),
]


def _tc_pre(h, wT, b, ai, aj, relp, wrT, br, ar):
    return pl.pallas_call(
        _tc_pre_body,
        grid=(NBLK,),
        in_specs=[_row_spec()] + _WEIGHT_SPECS,
        out_specs=_SCORE_OUT_SPECS,
        out_shape=_SCORE_OUT_SHAPES,
    )(h, wT, b, ai, aj, relp, wrT, br, ar)


_AGG_SPECS = [
    _row_spec(),
    pl.BlockSpec((NC, BLK, D), lambda i: (0, i, 0)),
    pl.BlockSpec((NT, BLK), lambda i: (0, i)),
    _full((1, D)),
    _full((1, D)),
]


def _tc_mid(h, accp, denp, g, be, wT, b, ai, aj, relp, wrT, br, ar):
    return pl.pallas_call(
        _tc_mid_body,
        grid=(NBLK,),
        in_specs=_AGG_SPECS + _WEIGHT_SPECS,
        out_specs=[_row_spec()] + _SCORE_OUT_SPECS,
        out_shape=[jax.ShapeDtypeStruct((NP, D), jnp.float32)] + _SCORE_OUT_SHAPES,
    )(h, accp, denp, g, be, wT, b, ai, aj, relp, wrT, br, ar)


def _tc_post(h, accp, denp, g, be):
    return pl.pallas_call(
        _tc_post_body,
        grid=(NBLK,),
        in_specs=_AGG_SPECS,
        out_specs=_row_spec(),
        out_shape=jax.ShapeDtypeStruct((NP, D), jnp.float32),
    )(h, accp, denp, g, be)


BB = 16
CHUNKS = EP // CH
NCH0 = 112
NCH1 = 48
CB1 = NS * NCH0


@functools.partial(
    pl.kernel,
    out_type=[
        jax.ShapeDtypeStruct((CHUNKS, CH), jnp.float32),
        jax.ShapeDtypeStruct((NT, NP), jnp.float32),
    ],
    mesh=plsc.VectorSubcoreMesh(core_axis_name="c", subcore_axis_name="s",
                                num_cores=NC, num_subcores=NS),
    compiler_params=pltpu.CompilerParams(needs_layout_passes=False),
    scratch_types=[
        pltpu.VMEM((BB, CH), jnp.int32),
        pltpu.VMEM((BB, CH), jnp.int32),
        pltpu.VMEM((BB, CH), jnp.int32),
        pltpu.VMEM((BB, CH), jnp.float32),
        pltpu.VMEM((NP,), jnp.float32),
        pltpu.VMEM((NP,), jnp.float32),
        pltpu.VMEM((RP,), jnp.float32),
        pltpu.VMEM((16,), jnp.float32),
        pltpu.VMEM((NP,), jnp.float32),
    ],
)
def _sc_weights(src_hbm, dst_hbm, typ_hbm, si_hbm, sj_hbm, sr_hbm, c_hbm,
                w_hbm, denp_hbm,
                srcb, dstb, typb, wb, siv, sjv, srv, cv, denv):
    cid = lax.axis_index("c")
    sid = lax.axis_index("s")
    wid = cid * NS + sid

    pltpu.sync_copy(si_hbm, siv)
    pltpu.sync_copy(sj_hbm, sjv)
    pltpu.sync_copy(sr_hbm, srv)
    pltpu.sync_copy(c_hbm, cv)

    zeros16 = jnp.zeros((16,), jnp.float32)

    def _zden(i, carry):
        denv[pl.ds(i * 16, 16)] = zeros16
        return carry
    lax.fori_loop(0, NP // 16, _zden, 0)

    cvec = cv[...]

    def _do(cbase, nch):
        for blk in range(nch // BB):
            bbase = cbase + blk * BB
            pltpu.sync_copy(src_hbm.at[pl.ds(bbase, BB)], srcb)
            pltpu.sync_copy(dst_hbm.at[pl.ds(bbase, BB)], dstb)
            pltpu.sync_copy(typ_hbm.at[pl.ds(bbase, BB)], typb)

            def _edge_scalar(r, carry):
                for q in range(8):
                    col = q * 16
                    d = dstb[r, pl.ds(col, 16)]
                    s = srcb[r, pl.ds(col, 16)]
                    t = typb[r, pl.ds(col, 16)]
                    gi = plsc.load_gather(siv, [d])
                    gj = plsc.load_gather(sjv, [s])
                    gr = plsc.load_gather(srv, [t])
                    tt = gi + gj + gr
                    a = jnp.where(tt > 0, tt, 0.2 * tt)
                    w = jnp.exp(a - cvec)
                    gid = (bbase + r) * CH + col + lax.iota(jnp.int32, 16)
                    w = jnp.where(gid < E, w, 0.0)
                    wb[r, pl.ds(col, 16)] = w
                    plsc.addupdate_scatter(denv, [d], w)
                return carry
            lax.fori_loop(0, BB, _edge_scalar, 0)
            pltpu.sync_copy(wb, w_hbm.at[pl.ds(bbase, BB)])

    @pl.when(cid == 0)
    def _():
        _do(sid * NCH0, NCH0)

    @pl.when(cid == 1)
    def _():
        _do(CB1 + sid * NCH1, NCH1)

    pltpu.sync_copy(denv, denp_hbm.at[wid])


@functools.partial(
    pl.kernel,
    out_type=jax.ShapeDtypeStruct((NC * NP, D), jnp.float32),
    mesh=plsc.VectorSubcoreMesh(core_axis_name="c", subcore_axis_name="s",
                                num_cores=NC, num_subcores=NS),
    compiler_params=pltpu.CompilerParams(needs_layout_passes=False),
    scratch_types=[
        pltpu.VMEM((BB, CH), jnp.int32),
        pltpu.VMEM((BB, CH), jnp.int32),
        pltpu.VMEM((BB, CH), jnp.float32),
        pltpu.VMEM((CH, D), jnp.float32),
        pltpu.VMEM((CH, D), jnp.float32),
        pltpu.VMEM_SHARED((NP, D), jnp.float32),
        pltpu.SemaphoreType.DMA,
        pltpu.SemaphoreType.DMA,
    ],
)
def _sc_rows(hx_hbm, src_hbm, dst_hbm, w_hbm, accp_hbm,
             srcb, dstb, wb, rows0, rows1, accs, sem0, sem1):
    cid = lax.axis_index("c")
    sid = lax.axis_index("s")
    wid = cid * NS + sid

    zeros16 = jnp.zeros((16,), jnp.float32)

    def _zrow(e, carry):
        for c in range(8):
            rows0[e, pl.ds(c * 16, 16)] = zeros16
        return carry
    lax.fori_loop(0, CH, _zrow, 0)

    row_base = sid * RPT
    for k in range(RPT // CH):
        pltpu.sync_copy(rows0, accs.at[pl.ds(row_base + k * CH, CH)])
    plsc.subcore_barrier()

    def _scale(rows, wrow):
        def _sc16(g, c2):
            wvec = wrow[pl.ds(g * 16, 16)]
            for l in range(16):
                ws = wvec[l]
                e = g * 16 + l
                for c in range(8):
                    sl = pl.ds(c * 16, 16)
                    rows[e, sl] = rows[e, sl] * ws
            return c2
        lax.fori_loop(0, CH // 16, _sc16, 0)

    def _do(cbase, nch):
        for blk in range(nch // BB):
            bbase = cbase + blk * BB
            pltpu.sync_copy(src_hbm.at[pl.ds(bbase, BB)], srcb)
            pltpu.sync_copy(dst_hbm.at[pl.ds(bbase, BB)], dstb)
            pltpu.sync_copy(w_hbm.at[pl.ds(bbase, BB)], wb)
            pltpu.async_copy(hx_hbm.at[srcb.at[0]], rows0, sem0)

            def _rowpair(jj, carry):
                j0 = 2 * jj
                pltpu.make_async_copy(hx_hbm.at[srcb.at[j0]], rows0, sem0).wait()
                pltpu.async_copy(hx_hbm.at[srcb.at[j0 + 1]], rows1, sem1)
                _scale(rows0, wb.at[j0])
                pltpu.sync_copy(rows0, accs.at[dstb.at[j0]], add=True)
                pltpu.make_async_copy(hx_hbm.at[srcb.at[j0 + 1]], rows1,
                                      sem1).wait()

                @pl.when(jj < BB // 2 - 1)
                def _():
                    pltpu.async_copy(hx_hbm.at[srcb.at[j0 + 2]], rows0, sem0)
                _scale(rows1, wb.at[j0 + 1])
                pltpu.sync_copy(rows1, accs.at[dstb.at[j0 + 1]], add=True)
                return carry
            lax.fori_loop(0, BB // 2, _rowpair, 0)

    @pl.when(cid == 0)
    def _():
        _do(sid * NCH0, NCH0)

    @pl.when(cid == 1)
    def _():
        _do(CB1 + sid * NCH1, NCH1)

    plsc.subcore_barrier()

    for k in range(RPT // CH):
        sl = pl.ds(row_base + k * CH, CH)
        pltpu.sync_copy(accs.at[sl], rows0)
        pltpu.sync_copy(rows0, accp_hbm.at[pl.ds(cid * NP + row_base + k * CH, CH)])


def _sc_agg(hx, src, dst, typ, si, sj, sr, c):
    w, denp = _sc_weights(src, dst, typ, si, sj, sr, c)
    accp = _sc_rows(hx, src, dst, w)
    return accp, denp


def _shift_const(bm, sr):
    bm3 = bm.reshape(NBLK, 8, D)
    c = jnp.max(bm3[:, 0, 0]) + jnp.max(bm3[:, 4, 0]) + jnp.max(sr)
    c = jnp.where(c > 0, c, 0.2 * c)
    return jnp.full((16,), c, jnp.float32)


def kernel(x, edge_index, edge_type, embedding,
           rel_emb1, W_lin1, b_lin1, W_rel1, b_rel1, attn1, gamma1, beta1,
           rel_emb2, W_lin2, b_lin2, W_rel2, b_rel2, attn2, gamma2, beta2):
    h = jnp.take(embedding, x, axis=0)
    h = jnp.pad(h, ((0, NP - N), (0, 0)))
    src = jnp.pad(edge_index[0], (0, EP - E)).reshape(CHUNKS, CH)
    dst = jnp.pad(edge_index[1], (0, EP - E)).reshape(CHUNKS, CH)
    typ = jnp.pad(edge_type, (0, EP - E)).reshape(CHUNKS, CH)
    relp1 = jnp.pad(rel_emb1, ((0, RP - R), (0, 0)))
    relp2 = jnp.pad(rel_emb2, ((0, RP - R), (0, 0)))

    def layer_weights(W_lin, b_lin, W_rel, b_rel, attn, relp):
        return (W_lin.T, b_lin[None], attn[:, :D], attn[:, D:2 * D],
                relp, W_rel.T, b_rel[None], attn[:, 2 * D:])

    w1 = layer_weights(W_lin1, b_lin1, W_rel1, b_rel1, attn1, relp1)
    w2 = layer_weights(W_lin2, b_lin2, W_rel2, b_rel2, attn2, relp2)

    hx1, si1, sj1, sr1, bm1 = _tc_pre(h, *w1)
    c1 = _shift_const(bm1, sr1)
    accp1, denp1 = _sc_agg(hx1, src, dst, typ,
                           si1.reshape(NP), sj1.reshape(NP),
                           sr1.reshape(RP), c1)

    h2, hx2, si2, sj2, sr2, bm2 = _tc_mid(
        h, accp1.reshape(NC, NP, D), denp1, gamma1[None], beta1[None], *w2)
    c2 = _shift_const(bm2, sr2)
    accp2, denp2 = _sc_agg(hx2, src, dst, typ,
                           si2.reshape(NP), sj2.reshape(NP),
                           sr2.reshape(RP), c2)

    hf = _tc_post(h2, accp2.reshape(NC, NP, D), denp2,
                  gamma2[None], beta2[None])
    return hf[:N]
